# baseline (speedup 1.0000x reference)
import functools

import jax
import jax.numpy as jnp
from jax import lax
from jax.experimental import pallas as pl
from jax.experimental.pallas import tpu as pltpu, tpu_sc as plsc

B, H, W, C = 2, 512, 512, 96
N = B * H * W
HW = H * W
NC, NS = 2, 16
NW = NC * NS
PW = N // NW
CHUNK = 64
NCH = PW // CHUNK
CG = C // 16
CP = 128


def _warp_body(table_hbm, fy_hbm, fx_hbm, out_hbm,
               fy_v, fx_v, ax_v, ay_v, idx_v, gbuf, out_v,
               sg0, sg1, so0, so1):
    wid = lax.axis_index("s") * NC + lax.axis_index("c")
    base = wid * PW
    gsems = (sg0, sg1)
    osems = (so0, so1)

    def stage(ci, b):
        pbase = base + ci * CHUNK
        row = lax.shift_right_logical(pbase, 9)
        y0 = jnp.bitwise_and(row, H - 1)
        bofs = lax.shift_left(lax.shift_right_logical(row, 9), 18)
        xb = jnp.bitwise_and(pbase, W - 1)

        pltpu.sync_copy(fy_hbm.at[pl.ds(pbase, CHUNK)], fy_v.at[b])
        pltpu.sync_copy(fx_hbm.at[pl.ds(pbase, CHUNK)], fx_v.at[b])

        y0f = y0.astype(jnp.float32)
        for g in range(CHUNK // 16):
            sl = pl.ds(g * 16, 16)
            qy = jnp.clip(y0f - fy_v[b, sl], 0.0, float(H - 1))
            iy = jnp.minimum(qy.astype(jnp.int32), H - 2)
            ay = qy - iy.astype(jnp.float32)
            xf = (xb + g * 16).astype(jnp.float32) + \
                lax.iota(jnp.int32, 16).astype(jnp.float32)
            qx = jnp.clip(xf - fx_v[b, sl], 0.0, float(W - 1))
            ix = jnp.minimum(qx.astype(jnp.int32), W - 2)
            ax = qx - ix.astype(jnp.float32)
            itl = bofs + lax.shift_left(iy, 9) + ix
            ay_v[b, sl] = ay
            ax_v[b, sl] = ax
            idx_v[b, 0, sl] = itl
            idx_v[b, 1, sl] = itl + 1
            idx_v[b, 2, sl] = itl + W
            idx_v[b, 3, sl] = itl + W + 1

        for q in range(4):
            pltpu.async_copy(table_hbm.at[idx_v.at[b, q]], gbuf.at[b, q],
                             gsems[b])

    def drain_gathers(b):
        for q in range(4):
            pltpu.make_async_copy(table_hbm.at[idx_v.at[b, q]],
                                  gbuf.at[b, q], gsems[b]).wait()

    def blend(ci, b):
        pbase = base + ci * CHUNK

        def p_body(p, c2):
            axb = ax_v[b, pl.ds(p, 16)][0]
            ayb = ay_v[b, pl.ds(p, 16)][0]
            for c in range(CG):
                csl = pl.ds(c * 16, 16)
                tl = gbuf[b, 0, p, csl]
                tr = gbuf[b, 1, p, csl]
                bl = gbuf[b, 2, p, csl]
                br = gbuf[b, 3, p, csl]
                top = tl + axb * (tr - tl)
                bot = bl + axb * (br - bl)
                out_v[b, p, csl] = top + ayb * (bot - top)
            return c2

        lax.fori_loop(0, CHUNK, p_body, 0)
        pltpu.async_copy(out_v.at[b], out_hbm.at[pl.ds(pbase, CHUNK)],
                         osems[b])

    def drain_out(b):
        pltpu.make_async_copy(out_v.at[b], out_hbm.at[pl.ds(base, CHUNK)],
                              osems[b]).wait()

    stage(0, 0)

    def pair_body(cp_i, carry):
        for b in range(2):
            ci = cp_i * 2 + b

            @pl.when(ci + 1 < NCH)
            def _():
                stage(ci + 1, 1 - b)

            drain_gathers(b)

            @pl.when(ci >= 2)
            def _():
                drain_out(b)

            blend(ci, b)
        return carry

    lax.fori_loop(0, NCH // 2, pair_body, 0)
    drain_out(0)
    drain_out(1)


@jax.jit
def kernel(image, flow):
    table = jnp.pad(image.reshape(N, C), ((0, 0), (0, CP - C)))
    fy = flow[..., 0].reshape(N)
    fx = flow[..., 1].reshape(N)

    mesh = plsc.VectorSubcoreMesh(core_axis_name="c", subcore_axis_name="s")
    out = pl.kernel(
        _warp_body,
        out_type=jax.ShapeDtypeStruct((N, C), jnp.float32),
        mesh=mesh,
        scratch_types=[
            pltpu.VMEM((2, CHUNK), jnp.float32),
            pltpu.VMEM((2, CHUNK), jnp.float32),
            pltpu.VMEM((2, CHUNK + 16), jnp.float32),
            pltpu.VMEM((2, CHUNK + 16), jnp.float32),
            pltpu.VMEM((2, 4, CHUNK), jnp.int32),
            pltpu.VMEM((2, 4, CHUNK, CP), jnp.float32),
            pltpu.VMEM((2, CHUNK, C), jnp.float32),
            pltpu.SemaphoreType.DMA,
            pltpu.SemaphoreType.DMA,
            pltpu.SemaphoreType.DMA,
            pltpu.SemaphoreType.DMA,
        ],
    )(table, fy, fx)
    return out.reshape(B, H, W, C)


# speedup vs baseline: 1.0980x; 1.0980x over previous
import functools

import jax
import jax.numpy as jnp
from jax import lax
from jax.experimental import pallas as pl
from jax.experimental.pallas import tpu as pltpu, tpu_sc as plsc

B, H, W, C = 2, 512, 512, 96
N = B * H * W
HW = H * W
NC, NS = 2, 16
NW = NC * NS
PW = N // NW
CHUNK = 64
NCH = PW // CHUNK
CG = C // 16
CP = 128


def _warp_body(table_hbm, fy_hbm, fx_hbm, out_hbm,
               fy_v, fx_v, ax_v, ay_v, idx_v, gbuf, out_v,
               sg0, sg1, so0, so1, sf0, sf1):
    wid = lax.axis_index("s") * NC + lax.axis_index("c")
    base = wid * PW
    gsems = (sg0, sg1)
    osems = (so0, so1)
    fsems = (sf0, sf1)

    def flow_fetch(ci, b):
        pbase = base + ci * CHUNK
        pltpu.async_copy(fy_hbm.at[pl.ds(pbase, CHUNK)], fy_v.at[b],
                         fsems[b])
        pltpu.async_copy(fx_hbm.at[pl.ds(pbase, CHUNK)], fx_v.at[b],
                         fsems[b])

    def stage(ci, b):
        pbase = base + ci * CHUNK
        row = lax.shift_right_logical(pbase, 9)
        y0 = jnp.bitwise_and(row, H - 1)
        bofs = lax.shift_left(lax.shift_right_logical(row, 9), 18)
        xb = jnp.bitwise_and(pbase, W - 1)

        pltpu.make_async_copy(fy_hbm.at[pl.ds(pbase, CHUNK)], fy_v.at[b],
                              fsems[b]).wait()
        pltpu.make_async_copy(fx_hbm.at[pl.ds(pbase, CHUNK)], fx_v.at[b],
                              fsems[b]).wait()

        y0f = y0.astype(jnp.float32)
        for g in range(CHUNK // 16):
            sl = pl.ds(g * 16, 16)
            qy = jnp.clip(y0f - fy_v[b, sl], 0.0, float(H - 1))
            iy = jnp.minimum(qy.astype(jnp.int32), H - 2)
            ay = qy - iy.astype(jnp.float32)
            xf = (xb + g * 16).astype(jnp.float32) + \
                lax.iota(jnp.int32, 16).astype(jnp.float32)
            qx = jnp.clip(xf - fx_v[b, sl], 0.0, float(W - 1))
            ix = jnp.minimum(qx.astype(jnp.int32), W - 2)
            ax = qx - ix.astype(jnp.float32)
            itl = bofs + lax.shift_left(iy, 9) + ix
            ay_v[b, sl] = ay
            ax_v[b, sl] = ax
            idx_v[b, 0, sl] = itl
            idx_v[b, 1, sl] = itl + 1
            idx_v[b, 2, sl] = itl + W
            idx_v[b, 3, sl] = itl + W + 1

        for q in range(4):
            pltpu.async_copy(table_hbm.at[idx_v.at[b, q]], gbuf.at[b, q],
                             gsems[b])

    def drain_gathers(b):
        for q in range(4):
            pltpu.make_async_copy(table_hbm.at[idx_v.at[b, q]],
                                  gbuf.at[b, q], gsems[b]).wait()

    def blend(ci, b):
        pbase = base + ci * CHUNK

        def p_body(p, c2):
            axb = ax_v[b, pl.ds(p, 16)][0]
            ayb = ay_v[b, pl.ds(p, 16)][0]
            for c in range(CG):
                csl = pl.ds(c * 16, 16)
                tl = gbuf[b, 0, p, csl]
                tr = gbuf[b, 1, p, csl]
                bl = gbuf[b, 2, p, csl]
                br = gbuf[b, 3, p, csl]
                top = tl + axb * (tr - tl)
                bot = bl + axb * (br - bl)
                out_v[b, p, csl] = top + ayb * (bot - top)
            return c2

        lax.fori_loop(0, CHUNK, p_body, 0)
        pltpu.async_copy(out_v.at[b], out_hbm.at[pl.ds(pbase, CHUNK)],
                         osems[b])

    def drain_out(b):
        pltpu.make_async_copy(out_v.at[b], out_hbm.at[pl.ds(base, CHUNK)],
                              osems[b]).wait()

    flow_fetch(0, 0)
    flow_fetch(1, 1)
    stage(0, 0)

    def pair_body(cp_i, carry):
        for b in range(2):
            ci = cp_i * 2 + b

            @pl.when(ci + 2 < NCH)
            def _():
                flow_fetch(ci + 2, b)

            @pl.when(ci + 1 < NCH)
            def _():
                stage(ci + 1, 1 - b)

            drain_gathers(b)

            @pl.when(ci >= 2)
            def _():
                drain_out(b)

            blend(ci, b)
        return carry

    lax.fori_loop(0, NCH // 2, pair_body, 0)
    drain_out(0)
    drain_out(1)


@jax.jit
def kernel(image, flow):
    table = jnp.pad(image.reshape(N, C), ((0, 0), (0, CP - C)))
    fy = flow[..., 0].reshape(N)
    fx = flow[..., 1].reshape(N)

    mesh = plsc.VectorSubcoreMesh(core_axis_name="c", subcore_axis_name="s")
    out = pl.kernel(
        _warp_body,
        out_type=jax.ShapeDtypeStruct((N, C), jnp.float32),
        mesh=mesh,
        scratch_types=[
            pltpu.VMEM((2, CHUNK), jnp.float32),
            pltpu.VMEM((2, CHUNK), jnp.float32),
            pltpu.VMEM((2, CHUNK + 16), jnp.float32),
            pltpu.VMEM((2, CHUNK + 16), jnp.float32),
            pltpu.VMEM((2, 4, CHUNK), jnp.int32),
            pltpu.VMEM((2, 4, CHUNK, CP), jnp.float32),
            pltpu.VMEM((2, CHUNK, C), jnp.float32),
            pltpu.SemaphoreType.DMA,
            pltpu.SemaphoreType.DMA,
            pltpu.SemaphoreType.DMA,
            pltpu.SemaphoreType.DMA,
            pltpu.SemaphoreType.DMA,
            pltpu.SemaphoreType.DMA,
        ],
    )(table, fy, fx)
    return out.reshape(B, H, W, C)
